# baseline (speedup 1.0000x reference)
import functools

import jax
import jax.numpy as jnp
from jax import lax
from jax.experimental import pallas as pl
from jax.experimental.pallas import tpu as pltpu
from jax.experimental.pallas import tpu_sc as plsc

NC = 2
NS = 16
NW = NC * NS
CHUNK = 128
DEG_W = 16
BN = 256
PB = 256


def _gather_rows_body(total, table_hbm, idx_hbm, out_hbm,
                      idx_a, idx_b, rows_a, rows_b, sem_a, sem_b):
    c = lax.axis_index("c")
    s = lax.axis_index("s")
    rpt = total // NW
    nchunks = rpt // CHUNK
    base = (c * NS + s) * rpt
    last = base + (nchunks - 1) * CHUNK

    pltpu.sync_copy(idx_hbm.at[pl.ds(base, CHUNK)], idx_a)
    pltpu.async_copy(table_hbm.at[idx_a], rows_a, sem_a)

    def pair_body(t, _):
        offb = base + (2 * t + 1) * CHUNK
        pltpu.sync_copy(idx_hbm.at[pl.ds(offb, CHUNK)], idx_b)
        pltpu.async_copy(table_hbm.at[idx_b], rows_b, sem_b)
        pltpu.make_async_copy(table_hbm.at[idx_a], rows_a, sem_a).wait()
        pltpu.sync_copy(rows_a,
                        out_hbm.at[pl.ds(base + 2 * t * CHUNK, CHUNK)])
        offa = jnp.minimum(base + (2 * t + 2) * CHUNK, last)
        pltpu.sync_copy(idx_hbm.at[pl.ds(offa, CHUNK)], idx_a)
        pltpu.async_copy(table_hbm.at[idx_a], rows_a, sem_a)
        pltpu.make_async_copy(table_hbm.at[idx_b], rows_b, sem_b).wait()
        pltpu.sync_copy(rows_b, out_hbm.at[pl.ds(offb, CHUNK)])
        return 0
    lax.fori_loop(0, nchunks // 2, pair_body, 0)
    pltpu.make_async_copy(table_hbm.at[idx_a], rows_a, sem_a).wait()


def _dense_body(x_ref, w_ref, b_ref, r_ref, dg_ref, p_ref, s_ref, dis_ref):
    h = lax.dot_general(x_ref[...], w_ref[...], (((1,), (1,)), ((), ())),
                        preferred_element_type=jnp.float32) + b_ref[...]
    deg = dg_ref[...] + 1.0
    dis = lax.rsqrt(deg)
    p_ref[...] = dis[:, :1] * jnp.maximum(h, 0.0)
    s_ref[...] = jnp.maximum(h + r_ref[...], 0.0) / deg[:, :1]
    dis_ref[...] = dis


def _prefix_body(nb, ms_ref, t_ref, carry):
    k = pl.program_id(0)

    @pl.when(k == 0)
    def _():
        carry[...] = jnp.zeros_like(carry)

    row = lax.broadcasted_iota(jnp.int32, (PB, PB), 0)
    col = lax.broadcasted_iota(jnp.int32, (PB, PB), 1)
    tri = jnp.where(row > col, 1.0, 0.0)
    blk = ms_ref[...]
    exc = lax.dot_general(tri, blk, (((1,), (0,)), ((), ())),
                          preferred_element_type=jnp.float32)
    t_ref[...] = exc + carry[...]
    carry[...] = carry[...] + jnp.sum(blk, axis=0, keepdims=True)


def _combine_body(te_ref, ts_ref, s_ref, dis_ref, out_ref):
    out_ref[...] = (dis_ref[...][:, :1] * (te_ref[...] - ts_ref[...])
                    + s_ref[...])


def kernel(x, edge_index, W, b, root_emb):
    N, D = x.shape
    E = edge_index.shape[1]

    n_unit = 2048
    n_pad = ((N + n_unit - 1) // n_unit) * n_unit
    if n_pad == N:
        n_pad += n_unit
    e_unit = NW * CHUNK
    e_pad = ((E + e_unit - 1) // e_unit) * e_unit
    gu = 2 * e_unit
    eg_pad = ((e_pad + gu - 1) // gu) * gu
    g_pad = ((n_pad + gu - 1) // gu) * gu
    trash = jnp.int32(n_pad - 1)

    src = edge_index[0].astype(jnp.int32)
    dst = edge_index[1].astype(jnp.int32)
    pad_e = e_pad - E
    src_p = jnp.concatenate([src, jnp.full((pad_e,), trash, jnp.int32)])
    dst_p = jnp.concatenate([dst, jnp.full((pad_e,), trash, jnp.int32)])
    x_p = jnp.pad(x, ((0, n_pad - N), (0, 0)))

    packed = jnp.sort((dst_p << 14) | src_p)
    ssrc = packed & jnp.int32(16383)
    bnd = jnp.searchsorted(
        packed, jnp.arange(n_pad + 1, dtype=jnp.int32) << 14)
    bnd = bnd.astype(jnp.int32)
    bnd_s = jnp.pad(bnd[:n_pad], (0, g_pad - n_pad))
    bnd_e = jnp.pad(bnd[1:n_pad + 1], (0, g_pad - n_pad))
    deg_cnt = (bnd[1:n_pad + 1] - bnd[:n_pad]).astype(jnp.float32)
    deg_rows = jnp.broadcast_to(deg_cnt[:, None], (n_pad, DEG_W))

    mesh = plsc.VectorSubcoreMesh(core_axis_name="c", subcore_axis_name="s")

    grid = n_pad // BN
    p_arr, s_arr, dis_arr = pl.pallas_call(
        _dense_body,
        grid=(grid,),
        in_specs=[
            pl.BlockSpec((BN, D), lambda i: (i, 0)),
            pl.BlockSpec((D, D), lambda i: (0, 0)),
            pl.BlockSpec((1, D), lambda i: (0, 0)),
            pl.BlockSpec((1, D), lambda i: (0, 0)),
            pl.BlockSpec((BN, DEG_W), lambda i: (i, 0)),
        ],
        out_specs=[
            pl.BlockSpec((BN, D), lambda i: (i, 0)),
            pl.BlockSpec((BN, D), lambda i: (i, 0)),
            pl.BlockSpec((BN, DEG_W), lambda i: (i, 0)),
        ],
        out_shape=[
            jax.ShapeDtypeStruct((n_pad, D), jnp.float32),
            jax.ShapeDtypeStruct((n_pad, D), jnp.float32),
            jax.ShapeDtypeStruct((n_pad, DEG_W), jnp.float32),
        ],
    )(x_p, W, b.reshape(1, D), root_emb, deg_rows)

    gather_edges = pl.kernel(
        functools.partial(_gather_rows_body, eg_pad),
        out_type=jax.ShapeDtypeStruct((eg_pad, D), jnp.float32),
        mesh=mesh,
        scratch_types=[
            pltpu.VMEM((CHUNK,), jnp.int32),
            pltpu.VMEM((CHUNK,), jnp.int32),
            pltpu.VMEM((CHUNK, D), jnp.float32),
            pltpu.VMEM((CHUNK, D), jnp.float32),
            pltpu.SemaphoreType.DMA,
            pltpu.SemaphoreType.DMA,
        ],
    )
    ms = gather_edges(p_arr, jnp.pad(ssrc, (0, eg_pad - e_pad)))

    nb = e_pad // PB
    t_arr = pl.pallas_call(
        functools.partial(_prefix_body, nb),
        grid=(nb + 1,),
        in_specs=[pl.BlockSpec((PB, D),
                               lambda k: (jnp.minimum(k, nb - 1), 0))],
        out_specs=pl.BlockSpec((PB, D), lambda k: (k, 0)),
        out_shape=jax.ShapeDtypeStruct((e_pad + PB, D), jnp.float32),
        scratch_shapes=[pltpu.VMEM((1, D), jnp.float32)],
    )(ms)

    gather_nodes = pl.kernel(
        functools.partial(_gather_rows_body, g_pad),
        out_type=jax.ShapeDtypeStruct((g_pad, D), jnp.float32),
        mesh=mesh,
        scratch_types=[
            pltpu.VMEM((CHUNK,), jnp.int32),
            pltpu.VMEM((CHUNK,), jnp.int32),
            pltpu.VMEM((CHUNK, D), jnp.float32),
            pltpu.VMEM((CHUNK, D), jnp.float32),
            pltpu.SemaphoreType.DMA,
            pltpu.SemaphoreType.DMA,
        ],
    )
    te = gather_nodes(t_arr, bnd_e)
    ts = gather_nodes(t_arr, bnd_s)

    out_pad = pl.pallas_call(
        _combine_body,
        grid=(grid,),
        in_specs=[
            pl.BlockSpec((BN, D), lambda i: (i, 0)),
            pl.BlockSpec((BN, D), lambda i: (i, 0)),
            pl.BlockSpec((BN, D), lambda i: (i, 0)),
            pl.BlockSpec((BN, DEG_W), lambda i: (i, 0)),
        ],
        out_specs=pl.BlockSpec((BN, D), lambda i: (i, 0)),
        out_shape=jax.ShapeDtypeStruct((n_pad, D), jnp.float32),
    )(te, ts, s_arr, dis_arr)

    return out_pad[:N]


# speedup vs baseline: 1.1830x; 1.1830x over previous
import functools

import jax
import jax.numpy as jnp
from jax import lax
from jax.experimental import pallas as pl
from jax.experimental.pallas import tpu as pltpu
from jax.experimental.pallas import tpu_sc as plsc

NC = 2
NS = 16
NW = NC * NS
CHUNK = 128
DEG_W = 16
BN = 256
PB = 256


def _gather_rows_body(total, table_hbm, idx_hbm, out_hbm, idx_v, rows_v, sem):
    c = lax.axis_index("c")
    s = lax.axis_index("s")
    rpt = total // NW
    nchunks = rpt // CHUNK
    base = (c * NS + s) * rpt

    def chunk_body(i, _):
        off = base + i * CHUNK
        pltpu.sync_copy(idx_hbm.at[pl.ds(off, CHUNK)], idx_v)
        pltpu.async_copy(table_hbm.at[idx_v], rows_v, sem).wait()
        pltpu.sync_copy(rows_v, out_hbm.at[pl.ds(off, CHUNK)])
        return 0
    lax.fori_loop(0, nchunks, chunk_body, 0)


def _dense_body(x_ref, w_ref, b_ref, r_ref, dg_ref, p_ref, s_ref, dis_ref):
    h = lax.dot_general(x_ref[...], w_ref[...], (((1,), (1,)), ((), ())),
                        preferred_element_type=jnp.float32) + b_ref[...]
    deg = dg_ref[...] + 1.0
    dis = lax.rsqrt(deg)
    p_ref[...] = dis[:, :1] * jnp.maximum(h, 0.0)
    s_ref[...] = jnp.maximum(h + r_ref[...], 0.0) / deg[:, :1]
    dis_ref[...] = dis


def _prefix_body(nb, ms_ref, t_ref, carry):
    k = pl.program_id(0)

    @pl.when(k == 0)
    def _():
        carry[...] = jnp.zeros_like(carry)

    row = lax.broadcasted_iota(jnp.int32, (PB, PB), 0)
    col = lax.broadcasted_iota(jnp.int32, (PB, PB), 1)
    tri = jnp.where(row > col, 1.0, 0.0)
    blk = ms_ref[...]
    exc = lax.dot_general(tri, blk, (((1,), (0,)), ((), ())),
                          preferred_element_type=jnp.float32)
    t_ref[...] = exc + carry[...]
    carry[...] = carry[...] + jnp.sum(blk, axis=0, keepdims=True)


def _combine_body(te_ref, ts_ref, s_ref, dis_ref, out_ref):
    out_ref[...] = (dis_ref[...][:, :1] * (te_ref[...] - ts_ref[...])
                    + s_ref[...])


def kernel(x, edge_index, W, b, root_emb):
    N, D = x.shape
    E = edge_index.shape[1]

    n_unit = 2048
    n_pad = ((N + n_unit - 1) // n_unit) * n_unit
    if n_pad == N:
        n_pad += n_unit
    e_unit = NW * CHUNK
    e_pad = ((E + e_unit - 1) // e_unit) * e_unit
    g_pad = ((n_pad + e_unit - 1) // e_unit) * e_unit
    trash = jnp.int32(n_pad - 1)

    src = edge_index[0].astype(jnp.int32)
    dst = edge_index[1].astype(jnp.int32)
    pad_e = e_pad - E
    src_p = jnp.concatenate([src, jnp.full((pad_e,), trash, jnp.int32)])
    dst_p = jnp.concatenate([dst, jnp.full((pad_e,), trash, jnp.int32)])
    x_p = jnp.pad(x, ((0, n_pad - N), (0, 0)))

    packed = jnp.sort((dst_p << 14) | src_p)
    ssrc = packed & jnp.int32(16383)
    bnd = jnp.searchsorted(
        packed, jnp.arange(n_pad + 1, dtype=jnp.int32) << 14)
    bnd = bnd.astype(jnp.int32)
    bnd_s = jnp.pad(bnd[:n_pad], (0, g_pad - n_pad))
    bnd_e = jnp.pad(bnd[1:n_pad + 1], (0, g_pad - n_pad))
    deg_cnt = (bnd[1:n_pad + 1] - bnd[:n_pad]).astype(jnp.float32)
    deg_rows = jnp.broadcast_to(deg_cnt[:, None], (n_pad, DEG_W))

    mesh = plsc.VectorSubcoreMesh(core_axis_name="c", subcore_axis_name="s")

    grid = n_pad // BN
    p_arr, s_arr, dis_arr = pl.pallas_call(
        _dense_body,
        grid=(grid,),
        in_specs=[
            pl.BlockSpec((BN, D), lambda i: (i, 0)),
            pl.BlockSpec((D, D), lambda i: (0, 0)),
            pl.BlockSpec((1, D), lambda i: (0, 0)),
            pl.BlockSpec((1, D), lambda i: (0, 0)),
            pl.BlockSpec((BN, DEG_W), lambda i: (i, 0)),
        ],
        out_specs=[
            pl.BlockSpec((BN, D), lambda i: (i, 0)),
            pl.BlockSpec((BN, D), lambda i: (i, 0)),
            pl.BlockSpec((BN, DEG_W), lambda i: (i, 0)),
        ],
        out_shape=[
            jax.ShapeDtypeStruct((n_pad, D), jnp.float32),
            jax.ShapeDtypeStruct((n_pad, D), jnp.float32),
            jax.ShapeDtypeStruct((n_pad, DEG_W), jnp.float32),
        ],
    )(x_p, W, b.reshape(1, D), root_emb, deg_rows)

    gather_edges = pl.kernel(
        functools.partial(_gather_rows_body, e_pad),
        out_type=jax.ShapeDtypeStruct((e_pad, D), jnp.float32),
        mesh=mesh,
        scratch_types=[
            pltpu.VMEM((CHUNK,), jnp.int32),
            pltpu.VMEM((CHUNK, D), jnp.float32),
            pltpu.SemaphoreType.DMA,
        ],
    )
    ms = gather_edges(p_arr, ssrc)

    nb = e_pad // PB
    t_arr = pl.pallas_call(
        functools.partial(_prefix_body, nb),
        grid=(nb + 1,),
        in_specs=[pl.BlockSpec((PB, D),
                               lambda k: (jnp.minimum(k, nb - 1), 0))],
        out_specs=pl.BlockSpec((PB, D), lambda k: (k, 0)),
        out_shape=jax.ShapeDtypeStruct((e_pad + PB, D), jnp.float32),
        scratch_shapes=[pltpu.VMEM((1, D), jnp.float32)],
    )(ms)

    gather_nodes = pl.kernel(
        functools.partial(_gather_rows_body, g_pad),
        out_type=jax.ShapeDtypeStruct((g_pad, D), jnp.float32),
        mesh=mesh,
        scratch_types=[
            pltpu.VMEM((CHUNK,), jnp.int32),
            pltpu.VMEM((CHUNK, D), jnp.float32),
            pltpu.SemaphoreType.DMA,
        ],
    )
    te = gather_nodes(t_arr, bnd_e)
    ts = gather_nodes(t_arr, bnd_s)

    out_pad = pl.pallas_call(
        _combine_body,
        grid=(grid,),
        in_specs=[
            pl.BlockSpec((BN, D), lambda i: (i, 0)),
            pl.BlockSpec((BN, D), lambda i: (i, 0)),
            pl.BlockSpec((BN, D), lambda i: (i, 0)),
            pl.BlockSpec((BN, DEG_W), lambda i: (i, 0)),
        ],
        out_specs=pl.BlockSpec((BN, D), lambda i: (i, 0)),
        out_shape=jax.ShapeDtypeStruct((n_pad, D), jnp.float32),
    )(te, ts, s_arr, dis_arr)

    return out_pad[:N]
